# baseline (speedup 1.0000x reference)
import jax
import jax.numpy as jnp
import numpy as np
from jax.experimental import pallas as pl
from jax.experimental.pallas import tpu as pltpu

_B = 256
_K = 10000
_J = 21
_D = 4
_IN = _J * _D
_KB = 2048
_NB = 5
_KP = _KB * _NB
_NN = 5
_BIG = 1e30


_BF = jnp.bfloat16


def _acos16(x):
    ax = jnp.minimum(jnp.abs(x), _BF(0.99609375))
    u = _BF(1.0) - ax
    s = u * jax.lax.rsqrt(u)
    p = _BF(0.046167117)
    p = p * ax - _BF(0.20157937)
    p = p * ax + _BF(1.5701435)
    r = s * p
    return jnp.where(x < 0, _BF(np.pi) - r, r)


def _kern(posej_ref, poseflat_ref, trt_ref, pen_ref, w0_ref, b0_ref,
          w1_ref, b1_ref, w2_ref, b2_ref, w3_ref, b3_ref, mmt_ref,
          out_ref, pn_scr, top5_scr):
    p = posej_ref[...]
    ss = jnp.sum(p * p, axis=2, keepdims=True)
    pn_scr[...] = (p * jax.lax.rsqrt(jnp.maximum(ss, 1e-24))).astype(_BF)

    top5_scr[...] = jnp.full((_B, 128), _BIG, jnp.float32)
    ids = jax.lax.broadcasted_iota(jnp.int32, (_B, _KB), 1) + 128

    def kblock(kb):
        dist = jnp.zeros((_B, _KB), jnp.float32)
        for j in range(_J):
            t = trt_ref[kb, j]
            pj = pn_scr[j]
            d = (pj[:, 0:1] * t[0:1, :] + pj[:, 1:2] * t[1:2, :]
                 + pj[:, 2:3] * t[2:3, :] + pj[:, 3:4] * t[3:4, :])
            dist = dist + _acos16(d).astype(jnp.float32)
        dist = dist + pen_ref[kb]

        kd = jax.lax.bitcast_convert_type(dist, jnp.int32)
        kd = jax.lax.bitcast_convert_type((kd & ~0xFFF) | ids,
                                          jnp.float32)
        cand = jnp.concatenate([top5_scr[...], kd], axis=1)
        for i in range(_NN):
            m = jnp.min(cand, axis=1, keepdims=True)
            cand = jnp.where(cand == m, _BIG, cand)
            mi = jax.lax.bitcast_convert_type(m, jnp.int32)
            top5_scr[:, i:i + 1] = jax.lax.bitcast_convert_type(
                (mi & ~0xFFF) | i, jnp.float32)

    for kb in range(_NB):
        kblock(kb)

    x = poseflat_ref[...]
    ssf = jnp.dot(x * x, mmt_ref[...], preferred_element_type=jnp.float32)
    xn = x * jax.lax.rsqrt(jnp.maximum(ssf, 1e-24))
    h = jnp.dot(xn, w0_ref[...], preferred_element_type=jnp.float32)
    h = jnp.maximum(h + b0_ref[...], 0.0)
    h = jnp.dot(h, w1_ref[...], preferred_element_type=jnp.float32)
    h = jnp.maximum(h + b1_ref[...], 0.0)
    h = jnp.dot(h, w2_ref[...], preferred_element_type=jnp.float32)
    h = jnp.maximum(h + b2_ref[...], 0.0)
    pred = jnp.dot(h, w3_ref[...], preferred_element_type=jnp.float32)
    pred = pred + b3_ref[...]

    lane = jax.lax.broadcasted_iota(jnp.int32, (_B, 128), 1)
    t5 = top5_scr[...]
    dv = jnp.sum(jnp.where(lane < _NN, t5, 0.0), axis=1,
                 keepdims=True) * (0.5 / _NN)
    out_ref[...] = jnp.sum(jnp.abs(pred - dv), axis=0,
                           keepdims=True) * (1.0 / _B)


@jax.jit
def kernel(pose, train_poses, W0, b0, W1, b1, W2, b2, W3, b3):
    posej = jnp.transpose(pose, (1, 0, 2))
    poseflat = pose.reshape(_B, _IN)
    t = jnp.transpose(train_poses, (1, 2, 0))
    t = jnp.pad(t, ((0, 0), (0, 0), (0, _KP - _K)))
    trt = jnp.transpose(t.reshape(_J, _D, _NB, _KB),
                        (2, 0, 1, 3)).astype(jnp.bfloat16)
    pen = jnp.where(jnp.arange(_KP, dtype=jnp.int32) < _K, 0.0,
                    _BIG).astype(jnp.float32).reshape(_NB, 1, _KB)
    mmt = jnp.asarray(np.kron(np.eye(_J, dtype=np.float32),
                              np.ones((_D, _D), dtype=np.float32)))
    out = pl.pallas_call(
        _kern,
        out_shape=jax.ShapeDtypeStruct((1, 1), jnp.float32),
        scratch_shapes=[
            pltpu.VMEM((_J, _B, _D), jnp.bfloat16),
            pltpu.VMEM((_B, 128), jnp.float32),
        ],
    )(posej, poseflat, trt, pen, W0, b0.reshape(1, -1), W1,
      b1.reshape(1, -1), W2, b2.reshape(1, -1), W3, b3.reshape(1, 1), mmt)
    return out[0, 0]


# speedup vs baseline: 1.0533x; 1.0533x over previous
import jax
import jax.numpy as jnp
import numpy as np
from jax.experimental import pallas as pl
from jax.experimental.pallas import tpu as pltpu

_B = 256
_K = 10000
_J = 21
_D = 4
_IN = _J * _D
_KB = 2048
_NB = 5
_KP = _KB * _NB
_NN = 5
_BIG = 1e30


_BF = jnp.bfloat16


def _acos16(x):
    ax = jnp.minimum(jnp.abs(x), _BF(0.99609375))
    u = _BF(1.0) - ax
    s = u * jax.lax.rsqrt(u)
    p = _BF(0.046167117)
    p = p * ax - _BF(0.20157937)
    p = p * ax + _BF(1.5701435)
    r = s * p
    return jnp.where(x < 0, _BF(np.pi) - r, r)


def _kern(posej_ref, poseflat_ref, trt_ref, pen_ref, w0_ref, b0_ref,
          w1_ref, b1_ref, w2_ref, b2_ref, w3_ref, b3_ref, mmt_ref,
          out_ref, pn_scr, top5_scr):
    p = posej_ref[...]
    ss = jnp.sum(p * p, axis=2, keepdims=True)
    pn_scr[...] = (p * jax.lax.rsqrt(jnp.maximum(ss, 1e-24))).astype(_BF)

    top5_scr[...] = jnp.full((_B, 128), _BIG, jnp.float32)
    ids = jax.lax.broadcasted_iota(jnp.int32, (_B, _KB), 1) + 128

    def kblock(kb, carry):
        def _dots(j):
            t = trt_ref[kb, j]
            pj = pn_scr[j]
            return (pj[:, 0:1] * t[0:1, :] + pj[:, 1:2] * t[1:2, :]
                    + pj[:, 2:3] * t[2:3, :] + pj[:, 3:4] * t[3:4, :])

        dist = _acos16(_dots(_J - 1)).astype(jnp.float32)
        for j in range(0, _J - 1, 2):
            rr = _acos16(_dots(j)) + _acos16(_dots(j + 1))
            dist = dist + rr.astype(jnp.float32)
        dist = dist + pen_ref[kb]

        kd = jax.lax.bitcast_convert_type(dist, jnp.int32)
        kd = jax.lax.bitcast_convert_type((kd & ~0xFFF) | ids,
                                          jnp.float32)
        cand = jnp.concatenate([top5_scr[...], kd], axis=1)
        for i in range(_NN):
            m = jnp.min(cand, axis=1, keepdims=True)
            cand = jnp.where(cand == m, _BIG, cand)
            mi = jax.lax.bitcast_convert_type(m, jnp.int32)
            top5_scr[:, i:i + 1] = jax.lax.bitcast_convert_type(
                (mi & ~0xFFF) | i, jnp.float32)
        return 0

    jax.lax.fori_loop(0, _NB, kblock, 0)

    x = poseflat_ref[...]
    ssf = jnp.dot(x * x, mmt_ref[...], preferred_element_type=jnp.float32)
    xn = x * jax.lax.rsqrt(jnp.maximum(ssf, 1e-24))
    h = jnp.dot(xn, w0_ref[...], preferred_element_type=jnp.float32)
    h = jnp.maximum(h + b0_ref[...], 0.0)
    h = jnp.dot(h, w1_ref[...], preferred_element_type=jnp.float32)
    h = jnp.maximum(h + b1_ref[...], 0.0)
    h = jnp.dot(h, w2_ref[...], preferred_element_type=jnp.float32)
    h = jnp.maximum(h + b2_ref[...], 0.0)
    pred = jnp.dot(h, w3_ref[...], preferred_element_type=jnp.float32)
    pred = pred + b3_ref[...]

    lane = jax.lax.broadcasted_iota(jnp.int32, (_B, 128), 1)
    t5 = top5_scr[...]
    dv = jnp.sum(jnp.where(lane < _NN, t5, 0.0), axis=1,
                 keepdims=True) * (0.5 / _NN)
    out_ref[...] = jnp.sum(jnp.abs(pred - dv), axis=0,
                           keepdims=True) * (1.0 / _B)


@jax.jit
def kernel(pose, train_poses, W0, b0, W1, b1, W2, b2, W3, b3):
    posej = jnp.transpose(pose, (1, 0, 2))
    poseflat = pose.reshape(_B, _IN)
    t = jnp.transpose(train_poses, (1, 2, 0))
    t = jnp.pad(t, ((0, 0), (0, 0), (0, _KP - _K)))
    trt = jnp.transpose(t.reshape(_J, _D, _NB, _KB),
                        (2, 0, 1, 3)).astype(jnp.bfloat16)
    pen = jnp.where(jnp.arange(_KP, dtype=jnp.int32) < _K, 0.0,
                    _BIG).astype(jnp.float32).reshape(_NB, 1, _KB)
    mmt = jnp.asarray(np.kron(np.eye(_J, dtype=np.float32),
                              np.ones((_D, _D), dtype=np.float32)))
    out = pl.pallas_call(
        _kern,
        out_shape=jax.ShapeDtypeStruct((1, 1), jnp.float32),
        scratch_shapes=[
            pltpu.VMEM((_J, _B, _D), jnp.bfloat16),
            pltpu.VMEM((_B, 128), jnp.float32),
        ],
    )(posej, poseflat, trt, pen, W0, b0.reshape(1, -1), W1,
      b1.reshape(1, -1), W2, b2.reshape(1, -1), W3, b3.reshape(1, 1), mmt)
    return out[0, 0]


# speedup vs baseline: 1.0639x; 1.0100x over previous
import jax
import jax.numpy as jnp
import numpy as np
from jax.experimental import pallas as pl
from jax.experimental.pallas import tpu as pltpu

_B = 256
_K = 10000
_J = 21
_D = 4
_IN = _J * _D
_KB = 2560
_NB = 4
_KP = _KB * _NB
_NN = 5
_BIG = 1e30


_BF = jnp.bfloat16


def _acos16(x):
    ax = jnp.minimum(jnp.abs(x), _BF(0.99609375))
    u = _BF(1.0) - ax
    s = u * jax.lax.rsqrt(u)
    p = _BF(0.046167117)
    p = p * ax - _BF(0.20157937)
    p = p * ax + _BF(1.5701435)
    r = s * p
    return jnp.where(x < 0, _BF(np.pi) - r, r)


def _kern(posej_ref, poseflat_ref, trt_ref, pen_ref, w0_ref, b0_ref,
          w1_ref, b1_ref, w2_ref, b2_ref, w3_ref, b3_ref, mmt_ref,
          out_ref, pn_scr, top5_scr):
    p = posej_ref[...]
    ss = jnp.sum(p * p, axis=2, keepdims=True)
    pn_scr[...] = (p * jax.lax.rsqrt(jnp.maximum(ss, 1e-24))).astype(_BF)

    top5_scr[...] = jnp.full((_B, 128), _BIG, jnp.float32)
    ids = jax.lax.broadcasted_iota(jnp.int32, (_B, _KB), 1) + 128

    def kblock(kb, carry):
        def _dots(j):
            t = trt_ref[kb, j]
            pj = pn_scr[j]
            return (pj[:, 0:1] * t[0:1, :] + pj[:, 1:2] * t[1:2, :]
                    + pj[:, 2:3] * t[2:3, :] + pj[:, 3:4] * t[3:4, :])

        dist = _acos16(_dots(_J - 1)).astype(jnp.float32)
        for j in range(0, _J - 1, 2):
            rr = _acos16(_dots(j)) + _acos16(_dots(j + 1))
            dist = dist + rr.astype(jnp.float32)
        dist = dist + pen_ref[kb]

        kd = jax.lax.bitcast_convert_type(dist, jnp.int32)
        kd = jax.lax.bitcast_convert_type((kd & ~0xFFF) | ids,
                                          jnp.float32)
        cand = jnp.concatenate([top5_scr[...], kd], axis=1)
        for i in range(_NN):
            m = jnp.min(cand, axis=1, keepdims=True)
            cand = jnp.where(cand == m, _BIG, cand)
            mi = jax.lax.bitcast_convert_type(m, jnp.int32)
            top5_scr[:, i:i + 1] = jax.lax.bitcast_convert_type(
                (mi & ~0xFFF) | i, jnp.float32)
        return 0

    jax.lax.fori_loop(0, _NB, kblock, 0)

    x = poseflat_ref[...]
    ssf = jnp.dot(x * x, mmt_ref[...], preferred_element_type=jnp.float32)
    xn = x * jax.lax.rsqrt(jnp.maximum(ssf, 1e-24))
    h = jnp.dot(xn, w0_ref[...], preferred_element_type=jnp.float32)
    h = jnp.maximum(h + b0_ref[...], 0.0)
    h = jnp.dot(h, w1_ref[...], preferred_element_type=jnp.float32)
    h = jnp.maximum(h + b1_ref[...], 0.0)
    h = jnp.dot(h, w2_ref[...], preferred_element_type=jnp.float32)
    h = jnp.maximum(h + b2_ref[...], 0.0)
    pred = jnp.dot(h, w3_ref[...], preferred_element_type=jnp.float32)
    pred = pred + b3_ref[...]

    lane = jax.lax.broadcasted_iota(jnp.int32, (_B, 128), 1)
    t5 = top5_scr[...]
    dv = jnp.sum(jnp.where(lane < _NN, t5, 0.0), axis=1,
                 keepdims=True) * (0.5 / _NN)
    out_ref[...] = jnp.sum(jnp.abs(pred - dv), axis=0,
                           keepdims=True) * (1.0 / _B)


@jax.jit
def kernel(pose, train_poses, W0, b0, W1, b1, W2, b2, W3, b3):
    posej = jnp.transpose(pose, (1, 0, 2))
    poseflat = pose.reshape(_B, _IN)
    t = jnp.transpose(train_poses, (1, 2, 0))
    t = jnp.pad(t, ((0, 0), (0, 0), (0, _KP - _K)))
    trt = jnp.transpose(t.reshape(_J, _D, _NB, _KB),
                        (2, 0, 1, 3)).astype(jnp.bfloat16)
    pen = jnp.where(jnp.arange(_KP, dtype=jnp.int32) < _K, 0.0,
                    _BIG).astype(jnp.float32).reshape(_NB, 1, _KB)
    mmt = jnp.asarray(np.kron(np.eye(_J, dtype=np.float32),
                              np.ones((_D, _D), dtype=np.float32)))
    out = pl.pallas_call(
        _kern,
        out_shape=jax.ShapeDtypeStruct((1, 1), jnp.float32),
        scratch_shapes=[
            pltpu.VMEM((_J, _B, _D), jnp.bfloat16),
            pltpu.VMEM((_B, 128), jnp.float32),
        ],
    )(posej, poseflat, trt, pen, W0, b0.reshape(1, -1), W1,
      b1.reshape(1, -1), W2, b2.reshape(1, -1), W3, b3.reshape(1, 1), mmt)
    return out[0, 0]
